# baseline (speedup 1.0000x reference)
import functools

import jax
import jax.numpy as jnp
from jax import lax
from jax.experimental import pallas as pl
from jax.experimental.pallas import tpu as pltpu
from jax.experimental.pallas import tpu_sc as plsc

NUM_EMB = 1000000
DIM = 32
BATCH = 16384
SEQ = 50
NC, NS = 2, 16
NW = NC * NS
BPW = BATCH // NW
IPW = BPW * SEQ
K = 128
JB = BPW // K
NCHUNK = SEQ * JB
NBUF = 8
NITER = NCHUNK // NBUF
L = 16
UNROLL = 4
TGROUPS = BPW // L
TSTEPS = SEQ * TGROUPS // UNROLL

_MESH = plsc.VectorSubcoreMesh(
    core_axis_name="c", subcore_axis_name="s", num_cores=NC, num_subcores=NS
)


@functools.partial(
    pl.kernel,
    out_type=jax.ShapeDtypeStruct((BATCH, SEQ, DIM), jnp.float32),
    mesh=_MESH,
    scratch_types=[
        pltpu.VMEM((IPW,), jnp.int32),
        pltpu.VMEM((IPW,), jnp.int32),
        pltpu.VMEM((NBUF, K, DIM), jnp.float32),
        pltpu.SemaphoreType.DMA((NBUF,)),
        pltpu.SemaphoreType.DMA((NBUF,)),
    ],
    compiler_params=pltpu.CompilerParams(
        use_tc_tiling_on_sc=False, needs_layout_passes=False
    ),
)
def _emb_lookup(tok_hbm, table_hbm, out_hbm, idx_v, idxt_v, rows_v, gsem, ssem):
    wid = lax.axis_index("s") * NC + lax.axis_index("c")
    b0 = wid * BPW
    pltpu.sync_copy(tok_hbm.at[pl.ds(wid * IPW, IPW)], idx_v)

    lanes = lax.iota(jnp.int32, L)

    def tbody(i, carry):
        for u in range(UNROLL):
            m = i * UNROLL + u
            t = m // TGROUPS
            s_base = (m % TGROUPS) * L
            vals = plsc.load_gather(idx_v, [s_base * SEQ + t + SEQ * lanes])
            idxt_v[pl.ds(t * BPW + s_base, L)] = vals
        return carry

    lax.fori_loop(0, TSTEPS, tbody, 0)

    def gather(c, b):
        return pltpu.make_async_copy(
            table_hbm.at[idxt_v.at[pl.ds(c * K, K)]], rows_v.at[b], gsem.at[b]
        )

    def store(c, b):
        t, j = c // JB, c % JB
        return pltpu.make_async_copy(
            rows_v.at[b], out_hbm.at[pl.ds(b0 + j * K, K), t], ssem.at[b]
        )

    for b in range(NBUF):
        gather(b, b).start()

    def body(it, carry):
        g = it * NBUF
        for b in range(NBUF):
            gather(g + b, b).wait()
            store(g + b, b).start()
        for b in range(NBUF):
            c_next = g + NBUF + b

            @pl.when(c_next < NCHUNK)
            def _():
                store(g + b, b).wait()
                gather(c_next, b).start()

        return carry

    lax.fori_loop(0, NITER, body, 0)

    for b in range(NBUF):
        store(NCHUNK - NBUF + b, b).wait()


def kernel(token_ids, weight):
    return _emb_lookup(token_ids.reshape(-1), weight)


# speedup vs baseline: 1.3038x; 1.3038x over previous
import functools

import jax
import jax.numpy as jnp
from jax import lax
from jax.experimental import pallas as pl
from jax.experimental.pallas import tpu as pltpu
from jax.experimental.pallas import tpu_sc as plsc

NUM_EMB = 1000000
DIM = 32
BATCH = 16384
SEQ = 50
NC, NS = 2, 16
NW = NC * NS
BPW = BATCH // NW
IPW = BPW * SEQ
K = 128
JB = BPW // K
NCHUNK = SEQ * JB
NBUF = 8
NITER = NCHUNK // NBUF
L = 16
UNROLL = 4
TGROUPS = BPW // L
TSTEPS = SEQ * TGROUPS // UNROLL

_MESH = plsc.VectorSubcoreMesh(
    core_axis_name="c", subcore_axis_name="s", num_cores=NC, num_subcores=NS
)


CS = DIM // 8
GB = BATCH // K


@functools.partial(
    pl.kernel,
    out_type=jax.ShapeDtypeStruct((SEQ, CS, GB, 8, K), jnp.float32),
    mesh=_MESH,
    scratch_types=[
        pltpu.VMEM((IPW,), jnp.int32),
        pltpu.VMEM((IPW,), jnp.int32),
        pltpu.VMEM((NBUF, K, DIM), jnp.float32),
        pltpu.VMEM((NBUF, CS, 8, K), jnp.float32),
        pltpu.SemaphoreType.DMA((NBUF,)),
        pltpu.SemaphoreType.DMA((NBUF,)),
    ],
    compiler_params=pltpu.CompilerParams(
        use_tc_tiling_on_sc=False, needs_layout_passes=False
    ),
)
def _emb_lookup(tok_hbm, table_hbm, out_hbm, idx_v, idxt_v, rows_v, tile_v,
                gsem, ssem):
    wid = lax.axis_index("s") * NC + lax.axis_index("c")
    pltpu.sync_copy(tok_hbm.at[pl.ds(wid * IPW, IPW)], idx_v)

    lanes = lax.iota(jnp.int32, L)

    def tbody(i, carry):
        for u in range(UNROLL):
            m = i * UNROLL + u
            t = m // TGROUPS
            s_base = (m % TGROUPS) * L
            vals = plsc.load_gather(idx_v, [s_base * SEQ + t + SEQ * lanes])
            idxt_v[pl.ds(t * BPW + s_base, L)] = vals
        return carry

    lax.fori_loop(0, TSTEPS, tbody, 0)

    def gather(c, b):
        return pltpu.make_async_copy(
            table_hbm.at[idxt_v.at[pl.ds(c * K, K)]], rows_v.at[b], gsem.at[b]
        )

    def store(c, b):
        t, j = c // JB, c % JB
        return pltpu.make_async_copy(
            tile_v.at[b], out_hbm.at[t, :, wid * JB + j], ssem.at[b]
        )

    rots = [(lanes + d) % L for d in range(L)]

    def transpose_chunk(b):
        def xbody(i, carry):
            l_vec = i * L + lanes
            for cb in range(DIM // L):
                for d in range(L):
                    col = cb * L + rots[d]
                    vals = plsc.load_gather(rows_v.at[b], [l_vec, col])
                    plsc.store_scatter(
                        tile_v.at[b],
                        [
                            lax.shift_right_logical(col, 3),
                            lax.bitwise_and(col, 7),
                            l_vec,
                        ],
                        vals,
                    )
            return carry

        lax.fori_loop(0, K // L, xbody, 0)

    for b in range(NBUF):
        gather(b, b).start()

    def body(it, carry):
        g = it * NBUF
        for b in range(NBUF):
            gather(g + b, b).wait()
            transpose_chunk(b)
            store(g + b, b).start()
        for b in range(NBUF):
            c_next = g + NBUF + b

            @pl.when(c_next < NCHUNK)
            def _():
                store(g + b, b).wait()
                gather(c_next, b).start()

        return carry

    lax.fori_loop(0, NITER, body, 0)

    for b in range(NBUF):
        store(NCHUNK - NBUF + b, b).wait()


def kernel(token_ids, weight):
    o5 = _emb_lookup(token_ids.reshape(-1), weight)
    return o5.transpose(2, 4, 0, 1, 3).reshape(BATCH, SEQ, DIM)
